# baseline (speedup 1.0000x reference)
import functools

import jax
import jax.numpy as jnp
from jax import lax
from jax.experimental import pallas as pl
from jax.experimental.pallas import tpu as pltpu
from jax.experimental.pallas import tpu_sc as plsc

N = 10000
E = 160000
F = 256
H = 128
NC = 2
NS = 16
NW = NC * NS
C = 128
NCHUNK = E // C
NPAD = 10240
RPS = N // NS
B = 1000

EPS = E // NS
CPSP = 80
CPSX = CPSP + 1
PADE = CPSP * C - EPS
NTR = 8


def _mesh():
    return plsc.VectorSubcoreMesh(
        core_axis_name="c", subcore_axis_name="s",
        num_cores=NC, num_subcores=NS)


SEG = NPAD // NS


def _sc_layer1(x, eidx):
    @functools.partial(
        pl.kernel,
        out_type=[
            jax.ShapeDtypeStruct((NC, N, H), jnp.float32),
            jax.ShapeDtypeStruct((NC * N, H), jnp.float32),
            jax.ShapeDtypeStruct((2, NS, 1, SEG), jnp.float32),
        ],
        mesh=_mesh(),
        scratch_types=[
            pltpu.VMEM_SHARED((N + NTR, H), jnp.float32),
            pltpu.VMEM_SHARED((NPAD,), jnp.float32),
            pltpu.VMEM((CPSX, 1, C), jnp.int32),
            [pltpu.VMEM((C,), jnp.int32) for _ in range(2)],
            [pltpu.VMEM((C, H), jnp.float32) for _ in range(2)],
            pltpu.VMEM((C,), jnp.float32),
            pltpu.VMEM((SEG,), jnp.float32),
            pltpu.VMEM((SEG,), jnp.float32),
            pltpu.VMEM((SEG,), jnp.float32),
            [pltpu.SemaphoreType.DMA for _ in range(2)],
            [pltpu.SemaphoreType.DMA for _ in range(2)],
            [pltpu.SemaphoreType.DMA for _ in range(2)],
            pltpu.SemaphoreType.DMA,
        ],
    )
    def l1(x_hbm, eidx_hbm, raw_hbm, xs_hbm, dinfo_hbm, accum, counts,
           didx, gidx, rows, ones_v, cnt_v, dis_v, inv_v,
           sem_g, sem_s, sem_i, sem_h):
        c = lax.axis_index("c")
        s = lax.axis_index("s")
        zero16 = jnp.zeros((16,), jnp.float32)
        one16 = jnp.ones((16,), jnp.float32)

        pltpu.sync_copy(eidx_hbm.at[2, s], didx)

        def ob(i, _):
            ones_v[pl.ds(i * 16, 16)] = one16
            return 0
        lax.fori_loop(0, C // 16, ob, 0)

        def zcb(i, _):
            cnt_v[pl.ds(i * 16, 16)] = zero16
            return 0
        lax.fori_loop(0, SEG // 16, zcb, 0)
        pltpu.sync_copy(cnt_v, counts.at[pl.ds(s * SEG, SEG)])
        plsc.subcore_barrier()

        def hb(t, _):
            pltpu.async_copy(ones_v, counts.at[didx.at[t, 0]], sem_h,
                             add=True)
            return 0
        lax.fori_loop(0, CPSP, hb, 0)

        def hd(t, _):
            pltpu.make_async_copy(ones_v, counts.at[pl.ds(0, C)],
                                  sem_h).wait()
            return 0
        lax.fori_loop(0, CPSP, hd, 0)
        plsc.subcore_barrier()

        pltpu.sync_copy(counts.at[pl.ds(s * SEG, SEG)], cnt_v)

        def db(i, _):
            deg = cnt_v[pl.ds(i * 16, 16)] + 1.0
            inv = 1.0 / deg
            inv_v[pl.ds(i * 16, 16)] = inv
            sq = 0.5 * (deg + 1.0)
            for _it in range(15):
                sq = 0.5 * (sq + deg / sq)
            dis_v[pl.ds(i * 16, 16)] = inv * sq
            return 0
        lax.fori_loop(0, SEG // 16, db, 0)
        pltpu.sync_copy(dis_v, dinfo_hbm.at[0, s, 0])
        pltpu.sync_copy(inv_v, dinfo_hbm.at[1, s, 0])

        def scale_block(nb, cnt):
            g0 = s * SEG + nb
            pltpu.sync_copy(x_hbm.at[pl.ds(g0, cnt), pl.ds(c * H, H)],
                            rows[0].at[pl.ds(0, cnt)])

            def nb_body(i, _):
                g = nb + i
                dis16 = dis_v[pl.ds((g // 16) * 16, 16)]
                idxv = lax.iota(jnp.int32, 16) * 0 + (g % 16)
                dsp = lax.gather(
                    dis16, idxv[:, None],
                    lax.GatherDimensionNumbers(
                        offset_dims=(), collapsed_slice_dims=(0,),
                        start_index_map=(0,)),
                    (1,), mode=lax.GatherScatterMode.PROMISE_IN_BOUNDS)
                for f in range(H // 16):
                    rows[0][i, pl.ds(f * 16, 16)] = (
                        rows[0][i, pl.ds(f * 16, 16)] * dsp)
                return 0
            lax.fori_loop(0, cnt, nb_body, 0)
            pltpu.sync_copy(rows[0].at[pl.ds(0, cnt)],
                            xs_hbm.at[pl.ds(c * N + g0, cnt)])

        @pl.when(s < NS - 1)
        def _scale_full():
            for q in range(SEG // C):
                scale_block(q * C, C)

        @pl.when(s == NS - 1)
        def _scale_tail():
            nlast = N - SEG * (NS - 1)
            for q in range(nlast // C):
                scale_block(q * C, C)
            scale_block((nlast // C) * C, nlast % C)

        def zb(t, _):
            rows[0][t // (H // 16), pl.ds((t % (H // 16)) * 16, 16)] = zero16
            return 0
        lax.fori_loop(0, C * (H // 16), zb, 0)
        r0 = s * RPS
        for q in range(RPS // C):
            pltpu.sync_copy(rows[0], accum.at[pl.ds(r0 + q * C, C)])
        pltpu.sync_copy(rows[0].at[pl.ds(0, RPS % C)],
                        accum.at[pl.ds(r0 + (RPS // C) * C, RPS % C)])

        @pl.when(s == 0)
        def _ztrash():
            pltpu.sync_copy(rows[0].at[pl.ds(0, NTR)],
                            accum.at[pl.ds(N, NTR)])

        plsc.subcore_barrier()

        def fire_i(k, t):
            pltpu.async_copy(eidx_hbm.at[c, s, t, 0], gidx[k], sem_i[k])

        def wait_i(k):
            pltpu.make_async_copy(eidx_hbm.at[c, s, 0, 0], gidx[k],
                                  sem_i[k]).wait()

        def fire_g(k):
            pltpu.async_copy(xs_hbm.at[gidx[k]], rows[k], sem_g[k])

        def wait_g(k):
            pltpu.make_async_copy(xs_hbm.at[pl.ds(0, C)], rows[k],
                                  sem_g[k]).wait()

        def fire_s(k, t):
            pltpu.async_copy(rows[k], accum.at[didx.at[t, 0]], sem_s[k],
                             add=True)

        def wait_s(k):
            pltpu.make_async_copy(rows[k], accum.at[pl.ds(0, C)],
                                  sem_s[k]).wait()

        fire_i(0, 0)
        fire_i(1, 1)
        wait_i(0)
        fire_g(0)
        wait_i(1)
        fire_g(1)
        wait_g(0)
        fire_s(0, 0)
        fire_i(0, 2)

        def step(t, k):
            wait_i(k)
            wait_s(k)
            fire_g(k)
            wait_g(1 - k)
            fire_s(1 - k, t - 1)
            fire_i(1 - k, t + 1)

        def body(i, _):
            t = 2 * i + 2
            step(t, 0)
            step(t + 1, 1)
            return 0
        lax.fori_loop(0, (CPSP - 2) // 2, body, 0)

        wait_g(1)
        fire_s(1, CPSP - 1)
        wait_s(0)
        wait_s(1)
        wait_i(0)

        plsc.subcore_barrier()
        o0 = s * SEG

        @pl.when(s < NS - 1)
        def _copy_main():
            pltpu.sync_copy(accum.at[pl.ds(o0, SEG)],
                            raw_hbm.at[c, pl.ds(o0, SEG)])

        @pl.when(s == NS - 1)
        def _copy_tail():
            pltpu.sync_copy(accum.at[pl.ds(o0, N - SEG * (NS - 1))],
                            raw_hbm.at[c, pl.ds(o0, N - SEG * (NS - 1))])

    return l1(x, eidx)


def _sc_aggregate(table2n, eidx):
    @functools.partial(
        pl.kernel,
        out_type=jax.ShapeDtypeStruct((NC, N, H), jnp.float32),
        mesh=_mesh(),
        scratch_types=[
            pltpu.VMEM_SHARED((N + NTR, H), jnp.float32),
            pltpu.VMEM((CPSX, 1, C), jnp.int32),
            [pltpu.VMEM((C,), jnp.int32) for _ in range(2)],
            [pltpu.VMEM((C, H), jnp.float32) for _ in range(2)],
            [pltpu.SemaphoreType.DMA for _ in range(2)],
            [pltpu.SemaphoreType.DMA for _ in range(2)],
            [pltpu.SemaphoreType.DMA for _ in range(2)],
        ],
    )
    def agg(tab_hbm, eidx_hbm, out_hbm, accum, didx, gidx, rows,
            sem_g, sem_s, sem_i):
        c = lax.axis_index("c")
        s = lax.axis_index("s")
        zero16 = jnp.zeros((16,), jnp.float32)

        pltpu.sync_copy(eidx_hbm.at[2, s], didx)

        def zb(t, _):
            rows[0][t // (H // 16), pl.ds((t % (H // 16)) * 16, 16)] = zero16
            return 0
        lax.fori_loop(0, C * (H // 16), zb, 0)
        r0 = s * RPS
        for q in range(RPS // C):
            pltpu.sync_copy(rows[0], accum.at[pl.ds(r0 + q * C, C)])
        pltpu.sync_copy(rows[0].at[pl.ds(0, RPS % C)],
                        accum.at[pl.ds(r0 + (RPS // C) * C, RPS % C)])
        @pl.when(s == 0)
        def _ztrash():
            pltpu.sync_copy(rows[0].at[pl.ds(0, NTR)],
                            accum.at[pl.ds(N, NTR)])

        def fire_i(k, t):
            pltpu.async_copy(eidx_hbm.at[c, s, t, 0], gidx[k], sem_i[k])

        def wait_i(k):
            pltpu.make_async_copy(eidx_hbm.at[c, s, 0, 0], gidx[k],
                                  sem_i[k]).wait()

        def fire_g(k):
            pltpu.async_copy(tab_hbm.at[gidx[k]], rows[k], sem_g[k])

        def wait_g(k):
            pltpu.make_async_copy(tab_hbm.at[pl.ds(0, C)], rows[k],
                                  sem_g[k]).wait()

        def fire_s(k, t):
            pltpu.async_copy(rows[k], accum.at[didx.at[t, 0]], sem_s[k],
                             add=True)

        def wait_s(k):
            pltpu.make_async_copy(rows[k], accum.at[pl.ds(0, C)],
                                  sem_s[k]).wait()

        fire_i(0, 0)
        fire_i(1, 1)
        plsc.subcore_barrier()
        wait_i(0)
        fire_g(0)
        wait_i(1)
        fire_g(1)
        wait_g(0)
        fire_s(0, 0)
        fire_i(0, 2)

        def step(t, k):
            wait_i(k)
            wait_s(k)
            fire_g(k)
            wait_g(1 - k)
            fire_s(1 - k, t - 1)
            fire_i(1 - k, t + 1)

        def body(i, _):
            t = 2 * i + 2
            step(t, 0)
            step(t + 1, 1)
            return 0
        lax.fori_loop(0, (CPSP - 2) // 2, body, 0)

        wait_g(1)
        fire_s(1, CPSP - 1)
        wait_s(0)
        wait_s(1)
        wait_i(0)

        plsc.subcore_barrier()
        o0 = s * 640

        @pl.when(s < NS - 1)
        def _copy_main():
            pltpu.sync_copy(accum.at[pl.ds(o0, 640)],
                            out_hbm.at[c, pl.ds(o0, 640)])

        @pl.when(s == NS - 1)
        def _copy_tail():
            pltpu.sync_copy(accum.at[pl.ds(o0, N - 640 * (NS - 1))],
                            out_hbm.at[c, pl.ds(o0, N - 640 * (NS - 1))])

    return agg(table2n, eidx)


def _tc_mlp(raw1, x, dis, inv, W1, b1, W2):
    def body(raw_ref, x_ref, dis_ref, inv_ref, W1_ref, b1_ref, W2_ref,
             z_ref, zs_ref):
        dis = dis_ref[...]
        agg = (jnp.concatenate([raw_ref[0], raw_ref[1]], axis=1) * dis
               + x_ref[...] * inv_ref[...])
        h = jnp.maximum(
            jnp.dot(agg, W1_ref[...], preferred_element_type=jnp.float32)
            + b1_ref[...], 0.0)
        z = jnp.dot(h, W2_ref[...], preferred_element_type=jnp.float32)
        z_ref[...] = z
        zs = z * dis
        zs_ref[0] = zs[:, :H]
        zs_ref[1] = zs[:, H:]

    return pl.pallas_call(
        body,
        grid=(N // B,),
        in_specs=[
            pl.BlockSpec((2, B, H), lambda i: (0, i, 0)),
            pl.BlockSpec((B, F), lambda i: (i, 0)),
            pl.BlockSpec((B, 1), lambda i: (i, 0)),
            pl.BlockSpec((B, 1), lambda i: (i, 0)),
            pl.BlockSpec((F, 512), lambda i: (0, 0)),
            pl.BlockSpec((1, 512), lambda i: (0, 0)),
            pl.BlockSpec((512, F), lambda i: (0, 0)),
        ],
        out_specs=[
            pl.BlockSpec((B, F), lambda i: (i, 0)),
            pl.BlockSpec((2, B, H), lambda i: (0, i, 0)),
        ],
        out_shape=[
            jax.ShapeDtypeStruct((N, F), jnp.float32),
            jax.ShapeDtypeStruct((2, N, H), jnp.float32),
        ],
    )(raw1, x, dis, inv, W1, b1, W2)


def _tc_decode(raw2, z, dis, inv, b2, Wd, bd):
    def body(raw_ref, z_ref, dis_ref, inv_ref, b2_ref, Wd_ref, bd_ref,
             lp_ref, emb_ref):
        out2 = (jnp.concatenate([raw_ref[0], raw_ref[1]], axis=1)
                * dis_ref[...] + z_ref[...] * inv_ref[...] + b2_ref[...])
        nrm = jnp.sqrt(jnp.sum(out2 * out2, axis=1, keepdims=True))
        emb = out2 / (nrm + 1e-12)
        emb_ref[...] = emb
        logits = (jnp.dot(emb, Wd_ref[...], preferred_element_type=jnp.float32)
                  + bd_ref[...])
        m = jnp.max(logits, axis=1, keepdims=True)
        lse = m + jnp.log(jnp.sum(jnp.exp(logits - m), axis=1, keepdims=True))
        lp_ref[...] = logits - lse

    return pl.pallas_call(
        body,
        grid=(N // B,),
        in_specs=[
            pl.BlockSpec((2, B, H), lambda i: (0, i, 0)),
            pl.BlockSpec((B, F), lambda i: (i, 0)),
            pl.BlockSpec((B, 1), lambda i: (i, 0)),
            pl.BlockSpec((B, 1), lambda i: (i, 0)),
            pl.BlockSpec((1, F), lambda i: (0, 0)),
            pl.BlockSpec((F, 128), lambda i: (0, 0)),
            pl.BlockSpec((1, 128), lambda i: (0, 0)),
        ],
        out_specs=[
            pl.BlockSpec((B, 128), lambda i: (i, 0)),
            pl.BlockSpec((B, F), lambda i: (i, 0)),
        ],
        out_shape=[
            jax.ShapeDtypeStruct((N, 128), jnp.float32),
            jax.ShapeDtypeStruct((N, F), jnp.float32),
        ],
    )(raw2, z, dis, inv, b2, Wd, bd)


def kernel(x, edge_index, W1, b1, W2, b2, Wd, bd):
    npad = CPSX * C - EPS
    srcr = edge_index[0].reshape(NS, EPS)
    dstr = edge_index[1].reshape(NS, EPS)
    gpad = jnp.broadcast_to((jnp.arange(npad, dtype=jnp.int32) * 131) % N,
                            (NS, npad))
    dpad = jnp.broadcast_to(N + (jnp.arange(npad, dtype=jnp.int32) % NTR),
                            (NS, npad))
    g0 = jnp.concatenate([srcr, gpad], axis=1)
    g1 = jnp.concatenate([srcr + N, gpad], axis=1)
    dd = jnp.concatenate([dstr, dpad], axis=1)
    eidx = jnp.stack([g0, g1, dd]).reshape(3, NS, CPSX, 1, C)
    raw1, _, dinfo = _sc_layer1(x, eidx)
    dis = dinfo[0].reshape(NPAD, 1)[:N]
    inv = dinfo[1].reshape(NPAD, 1)[:N]
    z, zs_cat = _tc_mlp(raw1, x, dis, inv, W1, b1.reshape(1, -1), W2)
    raw2 = _sc_aggregate(zs_cat.reshape(2 * N, H), eidx)
    lp, emb = _tc_decode(raw2, z, dis, inv, b2.reshape(1, -1), Wd,
                         bd.reshape(1, -1))
    return (lp, emb)


# speedup vs baseline: 1.0130x; 1.0130x over previous
import functools

import jax
import jax.numpy as jnp
from jax import lax
from jax.experimental import pallas as pl
from jax.experimental.pallas import tpu as pltpu
from jax.experimental.pallas import tpu_sc as plsc

N = 10000
E = 160000
F = 256
H = 128
NC = 2
NS = 16
NW = NC * NS
C = 128
NCHUNK = E // C
NPAD = 10240
RPS = N // NS
B = 1000

EPS = E // NS
CPSP = 80
CPSX = CPSP + 1
PADE = CPSP * C - EPS
NTR = 8


def _mesh():
    return plsc.VectorSubcoreMesh(
        core_axis_name="c", subcore_axis_name="s",
        num_cores=NC, num_subcores=NS)


SEG = NPAD // NS


def _sc_layer1(x, eidx):
    @functools.partial(
        pl.kernel,
        out_type=[
            jax.ShapeDtypeStruct((NC, N, H), jnp.float32),
            jax.ShapeDtypeStruct((NC * N, H), jnp.float32),
            jax.ShapeDtypeStruct((2, NS, 1, SEG), jnp.float32),
        ],
        mesh=_mesh(),
        scratch_types=[
            pltpu.VMEM_SHARED((N + NTR, H), jnp.float32),
            pltpu.VMEM_SHARED((NPAD,), jnp.float32),
            pltpu.VMEM((CPSX, 1, C), jnp.int32),
            [pltpu.VMEM((C,), jnp.int32) for _ in range(2)],
            [pltpu.VMEM((C, H), jnp.float32) for _ in range(2)],
            pltpu.VMEM((C,), jnp.float32),
            pltpu.VMEM((SEG,), jnp.float32),
            pltpu.VMEM((SEG,), jnp.float32),
            pltpu.VMEM((SEG,), jnp.float32),
            [pltpu.SemaphoreType.DMA for _ in range(2)],
            [pltpu.SemaphoreType.DMA for _ in range(2)],
            [pltpu.SemaphoreType.DMA for _ in range(2)],
            pltpu.SemaphoreType.DMA,
        ],
    )
    def l1(x_hbm, eidx_hbm, raw_hbm, xs_hbm, dinfo_hbm, accum, counts,
           didx, gidx, rows, ones_v, cnt_v, dis_v, inv_v,
           sem_g, sem_s, sem_i, sem_h):
        c = lax.axis_index("c")
        s = lax.axis_index("s")
        zero16 = jnp.zeros((16,), jnp.float32)
        one16 = jnp.ones((16,), jnp.float32)

        pltpu.sync_copy(eidx_hbm.at[2, s], didx)

        def ob(i, _):
            ones_v[pl.ds(i * 16, 16)] = one16
            return 0
        lax.fori_loop(0, C // 16, ob, 0)

        def zcb(i, _):
            cnt_v[pl.ds(i * 16, 16)] = zero16
            return 0
        lax.fori_loop(0, SEG // 16, zcb, 0)
        pltpu.sync_copy(cnt_v, counts.at[pl.ds(s * SEG, SEG)])
        plsc.subcore_barrier()

        def hb(t, _):
            pltpu.async_copy(ones_v, counts.at[didx.at[t, 0]], sem_h,
                             add=True)
            return 0
        lax.fori_loop(0, CPSP, hb, 0)

        def zb(t, _):
            rows[0][t // (H // 16), pl.ds((t % (H // 16)) * 16, 16)] = zero16
            return 0
        lax.fori_loop(0, C * (H // 16), zb, 0)
        r0 = s * RPS
        for q in range(RPS // C):
            pltpu.sync_copy(rows[0], accum.at[pl.ds(r0 + q * C, C)])
        pltpu.sync_copy(rows[0].at[pl.ds(0, RPS % C)],
                        accum.at[pl.ds(r0 + (RPS // C) * C, RPS % C)])

        @pl.when(s == 0)
        def _ztrash():
            pltpu.sync_copy(rows[0].at[pl.ds(0, NTR)],
                            accum.at[pl.ds(N, NTR)])

        def hd(t, _):
            pltpu.make_async_copy(ones_v, counts.at[pl.ds(0, C)],
                                  sem_h).wait()
            return 0
        lax.fori_loop(0, CPSP, hd, 0)
        plsc.subcore_barrier()

        pltpu.sync_copy(counts.at[pl.ds(s * SEG, SEG)], cnt_v)

        def db(i, _):
            deg = cnt_v[pl.ds(i * 16, 16)] + 1.0
            inv = 1.0 / deg
            inv_v[pl.ds(i * 16, 16)] = inv
            sq = 0.5 * (deg + 1.0)
            for _it in range(15):
                sq = 0.5 * (sq + deg / sq)
            dis_v[pl.ds(i * 16, 16)] = inv * sq
            return 0
        lax.fori_loop(0, SEG // 16, db, 0)
        pltpu.sync_copy(dis_v, dinfo_hbm.at[0, s, 0])
        pltpu.sync_copy(inv_v, dinfo_hbm.at[1, s, 0])

        def scale_block(nb, cnt):
            g0 = s * SEG + nb
            pltpu.sync_copy(x_hbm.at[pl.ds(g0, cnt), pl.ds(c * H, H)],
                            rows[0].at[pl.ds(0, cnt)])

            dnums = lax.GatherDimensionNumbers(
                offset_dims=(), collapsed_slice_dims=(0,),
                start_index_map=(0,))

            def grp_body(gi, _):
                dis16 = dis_v[pl.ds(nb + gi * 16, 16)]
                for j in range(16):
                    dsp = lax.gather(
                        dis16, jnp.full((16, 1), j, jnp.int32), dnums, (1,),
                        mode=lax.GatherScatterMode.PROMISE_IN_BOUNDS)
                    i = gi * 16 + j
                    for f in range(H // 16):
                        rows[0][i, pl.ds(f * 16, 16)] = (
                            rows[0][i, pl.ds(f * 16, 16)] * dsp)
                return 0
            lax.fori_loop(0, cnt // 16, grp_body, 0)
            pltpu.sync_copy(rows[0].at[pl.ds(0, cnt)],
                            xs_hbm.at[pl.ds(c * N + g0, cnt)])

        @pl.when(s < NS - 1)
        def _scale_full():
            for q in range(SEG // C):
                scale_block(q * C, C)

        @pl.when(s == NS - 1)
        def _scale_tail():
            nlast = N - SEG * (NS - 1)
            for q in range(nlast // C):
                scale_block(q * C, C)
            scale_block((nlast // C) * C, nlast % C)

        plsc.subcore_barrier()

        def fire_i(k, t):
            pltpu.async_copy(eidx_hbm.at[c, s, t, 0], gidx[k], sem_i[k])

        def wait_i(k):
            pltpu.make_async_copy(eidx_hbm.at[c, s, 0, 0], gidx[k],
                                  sem_i[k]).wait()

        def fire_g(k):
            pltpu.async_copy(xs_hbm.at[gidx[k]], rows[k], sem_g[k])

        def wait_g(k):
            pltpu.make_async_copy(xs_hbm.at[pl.ds(0, C)], rows[k],
                                  sem_g[k]).wait()

        def fire_s(k, t):
            pltpu.async_copy(rows[k], accum.at[didx.at[t, 0]], sem_s[k],
                             add=True)

        def wait_s(k):
            pltpu.make_async_copy(rows[k], accum.at[pl.ds(0, C)],
                                  sem_s[k]).wait()

        fire_i(0, 0)
        fire_i(1, 1)
        wait_i(0)
        fire_g(0)
        wait_i(1)
        fire_g(1)
        wait_g(0)
        fire_s(0, 0)
        fire_i(0, 2)

        def step(t, k):
            wait_i(k)
            wait_s(k)
            fire_g(k)
            wait_g(1 - k)
            fire_s(1 - k, t - 1)
            fire_i(1 - k, t + 1)

        def body(i, _):
            t = 2 * i + 2
            step(t, 0)
            step(t + 1, 1)
            return 0
        lax.fori_loop(0, (CPSP - 2) // 2, body, 0)

        wait_g(1)
        fire_s(1, CPSP - 1)
        wait_s(0)
        wait_s(1)
        wait_i(0)

        plsc.subcore_barrier()
        o0 = s * SEG

        @pl.when(s < NS - 1)
        def _copy_main():
            pltpu.sync_copy(accum.at[pl.ds(o0, SEG)],
                            raw_hbm.at[c, pl.ds(o0, SEG)])

        @pl.when(s == NS - 1)
        def _copy_tail():
            pltpu.sync_copy(accum.at[pl.ds(o0, N - SEG * (NS - 1))],
                            raw_hbm.at[c, pl.ds(o0, N - SEG * (NS - 1))])

    return l1(x, eidx)


def _sc_aggregate(table2n, eidx):
    @functools.partial(
        pl.kernel,
        out_type=jax.ShapeDtypeStruct((NC, N, H), jnp.float32),
        mesh=_mesh(),
        scratch_types=[
            pltpu.VMEM_SHARED((N + NTR, H), jnp.float32),
            pltpu.VMEM((CPSX, 1, C), jnp.int32),
            [pltpu.VMEM((C,), jnp.int32) for _ in range(2)],
            [pltpu.VMEM((C, H), jnp.float32) for _ in range(2)],
            [pltpu.SemaphoreType.DMA for _ in range(2)],
            [pltpu.SemaphoreType.DMA for _ in range(2)],
            [pltpu.SemaphoreType.DMA for _ in range(2)],
        ],
    )
    def agg(tab_hbm, eidx_hbm, out_hbm, accum, didx, gidx, rows,
            sem_g, sem_s, sem_i):
        c = lax.axis_index("c")
        s = lax.axis_index("s")
        zero16 = jnp.zeros((16,), jnp.float32)

        pltpu.sync_copy(eidx_hbm.at[2, s], didx)

        def zb(t, _):
            rows[0][t // (H // 16), pl.ds((t % (H // 16)) * 16, 16)] = zero16
            return 0
        lax.fori_loop(0, C * (H // 16), zb, 0)
        r0 = s * RPS
        for q in range(RPS // C):
            pltpu.sync_copy(rows[0], accum.at[pl.ds(r0 + q * C, C)])
        pltpu.sync_copy(rows[0].at[pl.ds(0, RPS % C)],
                        accum.at[pl.ds(r0 + (RPS // C) * C, RPS % C)])
        @pl.when(s == 0)
        def _ztrash():
            pltpu.sync_copy(rows[0].at[pl.ds(0, NTR)],
                            accum.at[pl.ds(N, NTR)])

        def fire_i(k, t):
            pltpu.async_copy(eidx_hbm.at[c, s, t, 0], gidx[k], sem_i[k])

        def wait_i(k):
            pltpu.make_async_copy(eidx_hbm.at[c, s, 0, 0], gidx[k],
                                  sem_i[k]).wait()

        def fire_g(k):
            pltpu.async_copy(tab_hbm.at[gidx[k]], rows[k], sem_g[k])

        def wait_g(k):
            pltpu.make_async_copy(tab_hbm.at[pl.ds(0, C)], rows[k],
                                  sem_g[k]).wait()

        def fire_s(k, t):
            pltpu.async_copy(rows[k], accum.at[didx.at[t, 0]], sem_s[k],
                             add=True)

        def wait_s(k):
            pltpu.make_async_copy(rows[k], accum.at[pl.ds(0, C)],
                                  sem_s[k]).wait()

        fire_i(0, 0)
        fire_i(1, 1)
        plsc.subcore_barrier()
        wait_i(0)
        fire_g(0)
        wait_i(1)
        fire_g(1)
        wait_g(0)
        fire_s(0, 0)
        fire_i(0, 2)

        def step(t, k):
            wait_i(k)
            wait_s(k)
            fire_g(k)
            wait_g(1 - k)
            fire_s(1 - k, t - 1)
            fire_i(1 - k, t + 1)

        def body(i, _):
            t = 2 * i + 2
            step(t, 0)
            step(t + 1, 1)
            return 0
        lax.fori_loop(0, (CPSP - 2) // 2, body, 0)

        wait_g(1)
        fire_s(1, CPSP - 1)
        wait_s(0)
        wait_s(1)
        wait_i(0)

        plsc.subcore_barrier()
        o0 = s * 640

        @pl.when(s < NS - 1)
        def _copy_main():
            pltpu.sync_copy(accum.at[pl.ds(o0, 640)],
                            out_hbm.at[c, pl.ds(o0, 640)])

        @pl.when(s == NS - 1)
        def _copy_tail():
            pltpu.sync_copy(accum.at[pl.ds(o0, N - 640 * (NS - 1))],
                            out_hbm.at[c, pl.ds(o0, N - 640 * (NS - 1))])

    return agg(table2n, eidx)


def _tc_mlp(raw1, x, dis, inv, W1, b1, W2):
    def body(raw_ref, x_ref, dis_ref, inv_ref, W1_ref, b1_ref, W2_ref,
             z_ref, zs_ref):
        dis = dis_ref[...]
        agg = (jnp.concatenate([raw_ref[0], raw_ref[1]], axis=1) * dis
               + x_ref[...] * inv_ref[...])
        h = jnp.maximum(
            jnp.dot(agg, W1_ref[...], preferred_element_type=jnp.float32)
            + b1_ref[...], 0.0)
        z = jnp.dot(h, W2_ref[...], preferred_element_type=jnp.float32)
        z_ref[...] = z
        zs = z * dis
        zs_ref[0] = zs[:, :H]
        zs_ref[1] = zs[:, H:]

    return pl.pallas_call(
        body,
        grid=(N // B,),
        in_specs=[
            pl.BlockSpec((2, B, H), lambda i: (0, i, 0)),
            pl.BlockSpec((B, F), lambda i: (i, 0)),
            pl.BlockSpec((B, 1), lambda i: (i, 0)),
            pl.BlockSpec((B, 1), lambda i: (i, 0)),
            pl.BlockSpec((F, 512), lambda i: (0, 0)),
            pl.BlockSpec((1, 512), lambda i: (0, 0)),
            pl.BlockSpec((512, F), lambda i: (0, 0)),
        ],
        out_specs=[
            pl.BlockSpec((B, F), lambda i: (i, 0)),
            pl.BlockSpec((2, B, H), lambda i: (0, i, 0)),
        ],
        out_shape=[
            jax.ShapeDtypeStruct((N, F), jnp.float32),
            jax.ShapeDtypeStruct((2, N, H), jnp.float32),
        ],
    )(raw1, x, dis, inv, W1, b1, W2)


def _tc_decode(raw2, z, dis, inv, b2, Wd, bd):
    def body(raw_ref, z_ref, dis_ref, inv_ref, b2_ref, Wd_ref, bd_ref,
             lp_ref, emb_ref):
        out2 = (jnp.concatenate([raw_ref[0], raw_ref[1]], axis=1)
                * dis_ref[...] + z_ref[...] * inv_ref[...] + b2_ref[...])
        nrm = jnp.sqrt(jnp.sum(out2 * out2, axis=1, keepdims=True))
        emb = out2 / (nrm + 1e-12)
        emb_ref[...] = emb
        logits = (jnp.dot(emb, Wd_ref[...], preferred_element_type=jnp.float32)
                  + bd_ref[...])
        m = jnp.max(logits, axis=1, keepdims=True)
        lse = m + jnp.log(jnp.sum(jnp.exp(logits - m), axis=1, keepdims=True))
        lp_ref[...] = logits - lse

    return pl.pallas_call(
        body,
        grid=(N // B,),
        in_specs=[
            pl.BlockSpec((2, B, H), lambda i: (0, i, 0)),
            pl.BlockSpec((B, F), lambda i: (i, 0)),
            pl.BlockSpec((B, 1), lambda i: (i, 0)),
            pl.BlockSpec((B, 1), lambda i: (i, 0)),
            pl.BlockSpec((1, F), lambda i: (0, 0)),
            pl.BlockSpec((F, 128), lambda i: (0, 0)),
            pl.BlockSpec((1, 128), lambda i: (0, 0)),
        ],
        out_specs=[
            pl.BlockSpec((B, 128), lambda i: (i, 0)),
            pl.BlockSpec((B, F), lambda i: (i, 0)),
        ],
        out_shape=[
            jax.ShapeDtypeStruct((N, 128), jnp.float32),
            jax.ShapeDtypeStruct((N, F), jnp.float32),
        ],
    )(raw2, z, dis, inv, b2, Wd, bd)


def kernel(x, edge_index, W1, b1, W2, b2, Wd, bd):
    npad = CPSX * C - EPS
    srcr = edge_index[0].reshape(NS, EPS)
    dstr = edge_index[1].reshape(NS, EPS)
    gpad = jnp.broadcast_to((jnp.arange(npad, dtype=jnp.int32) * 131) % N,
                            (NS, npad))
    dpad = jnp.broadcast_to(N + (jnp.arange(npad, dtype=jnp.int32) % NTR),
                            (NS, npad))
    g0 = jnp.concatenate([srcr, gpad], axis=1)
    g1 = jnp.concatenate([srcr + N, gpad], axis=1)
    dd = jnp.concatenate([dstr, dpad], axis=1)
    eidx = jnp.stack([g0, g1, dd]).reshape(3, NS, CPSX, 1, C)
    raw1, _, dinfo = _sc_layer1(x, eidx)
    dis = dinfo[0].reshape(NPAD, 1)[:N]
    inv = dinfo[1].reshape(NPAD, 1)[:N]
    z, zs_cat = _tc_mlp(raw1, x, dis, inv, W1, b1.reshape(1, -1), W2)
    raw2 = _sc_aggregate(zs_cat.reshape(2 * N, H), eidx)
    lp, emb = _tc_decode(raw2, z, dis, inv, b2.reshape(1, -1), Wd,
                         bd.reshape(1, -1))
    return (lp, emb)


# speedup vs baseline: 1.0131x; 1.0001x over previous
import functools

import jax
import jax.numpy as jnp
from jax import lax
from jax.experimental import pallas as pl
from jax.experimental.pallas import tpu as pltpu
from jax.experimental.pallas import tpu_sc as plsc

N = 10000
E = 160000
F = 256
H = 128
NC = 2
NS = 16
NW = NC * NS
C = 128
NCHUNK = E // C
NPAD = 10240
RPS = N // NS
B = 1000

EPS = E // NS
CPSP = 80
CPSX = CPSP + 1
PADE = CPSP * C - EPS
NTR = 8


def _mesh():
    return plsc.VectorSubcoreMesh(
        core_axis_name="c", subcore_axis_name="s",
        num_cores=NC, num_subcores=NS)


SEG = NPAD // NS


def _sc_layer1(x, eidx):
    @functools.partial(
        pl.kernel,
        out_type=[
            jax.ShapeDtypeStruct((NC, N, H), jnp.float32),
            jax.ShapeDtypeStruct((NC * N, H), jnp.float32),
            jax.ShapeDtypeStruct((2, NS, 1, SEG), jnp.float32),
        ],
        mesh=_mesh(),
        scratch_types=[
            pltpu.VMEM_SHARED((N + NTR, H), jnp.float32),
            pltpu.VMEM_SHARED((NPAD,), jnp.float32),
            pltpu.VMEM((CPSX, 1, C), jnp.int32),
            [pltpu.VMEM((C,), jnp.int32) for _ in range(2)],
            [pltpu.VMEM((C, H), jnp.float32) for _ in range(2)],
            pltpu.VMEM((C,), jnp.float32),
            pltpu.VMEM((SEG,), jnp.float32),
            pltpu.VMEM((SEG,), jnp.float32),
            pltpu.VMEM((SEG,), jnp.float32),
            [pltpu.SemaphoreType.DMA for _ in range(2)],
            [pltpu.SemaphoreType.DMA for _ in range(2)],
            [pltpu.SemaphoreType.DMA for _ in range(2)],
            pltpu.SemaphoreType.DMA,
        ],
    )
    def l1(x_hbm, eidx_hbm, raw_hbm, xs_hbm, dinfo_hbm, accum, counts,
           didx, gidx, rows, ones_v, cnt_v, dis_v, inv_v,
           sem_g, sem_s, sem_i, sem_h):
        c = lax.axis_index("c")
        s = lax.axis_index("s")
        zero16 = jnp.zeros((16,), jnp.float32)
        one16 = jnp.ones((16,), jnp.float32)

        pltpu.sync_copy(eidx_hbm.at[2, s], didx)

        def ob(i, _):
            ones_v[pl.ds(i * 16, 16)] = one16
            return 0
        lax.fori_loop(0, C // 16, ob, 0)

        def zcb(i, _):
            cnt_v[pl.ds(i * 16, 16)] = zero16
            return 0
        lax.fori_loop(0, SEG // 16, zcb, 0)
        pltpu.sync_copy(cnt_v, counts.at[pl.ds(s * SEG, SEG)])
        plsc.subcore_barrier()

        def hb(t, _):
            pltpu.async_copy(ones_v, counts.at[didx.at[t, 0]], sem_h,
                             add=True)
            return 0
        lax.fori_loop(0, CPSP, hb, 0)

        def zb(t, _):
            rows[0][t // (H // 16), pl.ds((t % (H // 16)) * 16, 16)] = zero16
            return 0
        lax.fori_loop(0, C * (H // 16), zb, 0)
        r0 = s * RPS
        for q in range(RPS // C):
            pltpu.sync_copy(rows[0], accum.at[pl.ds(r0 + q * C, C)])
        pltpu.sync_copy(rows[0].at[pl.ds(0, RPS % C)],
                        accum.at[pl.ds(r0 + (RPS // C) * C, RPS % C)])

        @pl.when(s == 0)
        def _ztrash():
            pltpu.sync_copy(rows[0].at[pl.ds(0, NTR)],
                            accum.at[pl.ds(N, NTR)])

        def hd(t, _):
            pltpu.make_async_copy(ones_v, counts.at[pl.ds(0, C)],
                                  sem_h).wait()
            return 0
        lax.fori_loop(0, CPSP, hd, 0)
        plsc.subcore_barrier()

        pltpu.sync_copy(counts.at[pl.ds(s * SEG, SEG)], cnt_v)

        def db(i, _):
            deg = cnt_v[pl.ds(i * 16, 16)] + 1.0
            inv = 1.0 / deg
            inv_v[pl.ds(i * 16, 16)] = inv
            sq = 0.5 * (deg + 1.0)
            for _it in range(15):
                sq = 0.5 * (sq + deg / sq)
            dis_v[pl.ds(i * 16, 16)] = inv * sq
            return 0
        lax.fori_loop(0, SEG // 16, db, 0)
        pltpu.sync_copy(dis_v, dinfo_hbm.at[0, s, 0])
        pltpu.sync_copy(inv_v, dinfo_hbm.at[1, s, 0])

        def scale_block(nb, cnt):
            g0 = s * SEG + nb
            pltpu.sync_copy(x_hbm.at[pl.ds(g0, cnt), pl.ds(c * H, H)],
                            rows[0].at[pl.ds(0, cnt)])

            dnums = lax.GatherDimensionNumbers(
                offset_dims=(), collapsed_slice_dims=(0,),
                start_index_map=(0,))

            def grp_body(gi, _):
                dis16 = dis_v[pl.ds(nb + gi * 16, 16)]
                for j in range(16):
                    dsp = lax.gather(
                        dis16, jnp.full((16, 1), j, jnp.int32), dnums, (1,),
                        mode=lax.GatherScatterMode.PROMISE_IN_BOUNDS)
                    i = gi * 16 + j
                    for f in range(H // 16):
                        rows[0][i, pl.ds(f * 16, 16)] = (
                            rows[0][i, pl.ds(f * 16, 16)] * dsp)
                return 0
            lax.fori_loop(0, cnt // 16, grp_body, 0)
            pltpu.sync_copy(rows[0].at[pl.ds(0, cnt)],
                            xs_hbm.at[pl.ds(c * N + g0, cnt)])

        @pl.when(s < NS - 1)
        def _scale_full():
            for q in range(SEG // C):
                scale_block(q * C, C)

        @pl.when(s == NS - 1)
        def _scale_tail():
            nlast = N - SEG * (NS - 1)
            for q in range(nlast // C):
                scale_block(q * C, C)
            scale_block((nlast // C) * C, nlast % C)

        plsc.subcore_barrier()

        def fire_i(k, t):
            pltpu.async_copy(eidx_hbm.at[c, s, t, 0], gidx[k], sem_i[k])

        def wait_i(k):
            pltpu.make_async_copy(eidx_hbm.at[c, s, 0, 0], gidx[k],
                                  sem_i[k]).wait()

        def fire_g(k):
            pltpu.async_copy(xs_hbm.at[gidx[k]], rows[k], sem_g[k])

        def wait_g(k):
            pltpu.make_async_copy(xs_hbm.at[pl.ds(0, C)], rows[k],
                                  sem_g[k]).wait()

        def fire_s(k, t):
            pltpu.async_copy(rows[k], accum.at[didx.at[t, 0]], sem_s[k],
                             add=True)

        def wait_s(k):
            pltpu.make_async_copy(rows[k], accum.at[pl.ds(0, C)],
                                  sem_s[k]).wait()

        fire_i(0, 0)
        fire_i(1, 1)
        wait_i(0)
        fire_g(0)
        wait_i(1)
        fire_g(1)
        wait_g(0)
        fire_s(0, 0)
        fire_i(0, 2)

        def step(t, k):
            wait_i(k)
            wait_s(k)
            fire_g(k)
            wait_g(1 - k)
            fire_s(1 - k, t - 1)
            fire_i(1 - k, t + 1)

        def body(i, _):
            t = 2 * i + 2
            step(t, 0)
            step(t + 1, 1)
            return 0
        lax.fori_loop(0, (CPSP - 2) // 2, body, 0)

        wait_g(1)
        fire_s(1, CPSP - 1)
        wait_s(0)
        wait_s(1)
        wait_i(0)

        plsc.subcore_barrier()
        o0 = s * SEG

        @pl.when(s < NS - 1)
        def _copy_main():
            pltpu.sync_copy(accum.at[pl.ds(o0, SEG)],
                            raw_hbm.at[c, pl.ds(o0, SEG)])

        @pl.when(s == NS - 1)
        def _copy_tail():
            pltpu.sync_copy(accum.at[pl.ds(o0, N - SEG * (NS - 1))],
                            raw_hbm.at[c, pl.ds(o0, N - SEG * (NS - 1))])

    return l1(x, eidx)


def _sc_aggregate(table2n, eidx):
    @functools.partial(
        pl.kernel,
        out_type=jax.ShapeDtypeStruct((NC, N, H), jnp.float32),
        mesh=_mesh(),
        scratch_types=[
            pltpu.VMEM_SHARED((N + NTR, H), jnp.float32),
            pltpu.VMEM((CPSX, 1, C), jnp.int32),
            [pltpu.VMEM((C,), jnp.int32) for _ in range(2)],
            [pltpu.VMEM((C, H), jnp.float32) for _ in range(2)],
            [pltpu.SemaphoreType.DMA for _ in range(2)],
            [pltpu.SemaphoreType.DMA for _ in range(2)],
            [pltpu.SemaphoreType.DMA for _ in range(2)],
        ],
    )
    def agg(tab_hbm, eidx_hbm, out_hbm, accum, didx, gidx, rows,
            sem_g, sem_s, sem_i):
        c = lax.axis_index("c")
        s = lax.axis_index("s")
        zero16 = jnp.zeros((16,), jnp.float32)

        pltpu.sync_copy(eidx_hbm.at[2, s], didx)

        def zb(t, _):
            rows[0][t // (H // 16), pl.ds((t % (H // 16)) * 16, 16)] = zero16
            return 0
        lax.fori_loop(0, C * (H // 16), zb, 0)
        r0 = s * RPS
        for q in range(RPS // C):
            pltpu.sync_copy(rows[0], accum.at[pl.ds(r0 + q * C, C)])
        pltpu.sync_copy(rows[0].at[pl.ds(0, RPS % C)],
                        accum.at[pl.ds(r0 + (RPS // C) * C, RPS % C)])
        @pl.when(s == 0)
        def _ztrash():
            pltpu.sync_copy(rows[0].at[pl.ds(0, NTR)],
                            accum.at[pl.ds(N, NTR)])

        def fire_i(k, t):
            pltpu.async_copy(eidx_hbm.at[c, s, t, 0], gidx[k], sem_i[k])

        def wait_i(k):
            pltpu.make_async_copy(eidx_hbm.at[c, s, 0, 0], gidx[k],
                                  sem_i[k]).wait()

        def fire_g(k):
            pltpu.async_copy(tab_hbm.at[gidx[k]], rows[k], sem_g[k])

        def wait_g(k):
            pltpu.make_async_copy(tab_hbm.at[pl.ds(0, C)], rows[k],
                                  sem_g[k]).wait()

        def fire_s(k, t):
            pltpu.async_copy(rows[k], accum.at[didx.at[t, 0]], sem_s[k],
                             add=True)

        def wait_s(k):
            pltpu.make_async_copy(rows[k], accum.at[pl.ds(0, C)],
                                  sem_s[k]).wait()

        fire_i(0, 0)
        fire_i(1, 1)
        plsc.subcore_barrier()
        wait_i(0)
        fire_g(0)
        wait_i(1)
        fire_g(1)
        wait_g(0)
        fire_s(0, 0)
        fire_i(0, 2)

        def step(t, k):
            wait_i(k)
            wait_s(k)
            fire_g(k)
            wait_g(1 - k)
            fire_s(1 - k, t - 1)
            fire_i(1 - k, t + 1)

        def body(i, _):
            t = 2 * i + 2
            step(t, 0)
            step(t + 1, 1)
            return 0
        lax.fori_loop(0, (CPSP - 2) // 2, body, 0)

        wait_g(1)
        fire_s(1, CPSP - 1)
        wait_s(0)
        wait_s(1)
        wait_i(0)

        plsc.subcore_barrier()
        o0 = s * 640

        @pl.when(s < NS - 1)
        def _copy_main():
            pltpu.sync_copy(accum.at[pl.ds(o0, 640)],
                            out_hbm.at[c, pl.ds(o0, 640)])

        @pl.when(s == NS - 1)
        def _copy_tail():
            pltpu.sync_copy(accum.at[pl.ds(o0, N - 640 * (NS - 1))],
                            out_hbm.at[c, pl.ds(o0, N - 640 * (NS - 1))])

    return agg(table2n, eidx)


def _tc_mlp(raw1, x, dis, inv, W1, b1, W2):
    def body(raw_ref, x_ref, dis_ref, inv_ref, W1_ref, b1_ref, W2_ref,
             z_ref, zs_ref):
        dis = dis_ref[...]
        agg = (jnp.concatenate([raw_ref[0], raw_ref[1]], axis=1) * dis
               + x_ref[...] * inv_ref[...])
        h = jnp.maximum(
            jnp.dot(agg.astype(jnp.bfloat16),
                    W1_ref[...].astype(jnp.bfloat16),
                    preferred_element_type=jnp.float32)
            + b1_ref[...], 0.0)
        z = jnp.dot(h.astype(jnp.bfloat16), W2_ref[...].astype(jnp.bfloat16),
                    preferred_element_type=jnp.float32)
        z_ref[...] = z
        zs = z * dis
        zs_ref[0] = zs[:, :H]
        zs_ref[1] = zs[:, H:]

    return pl.pallas_call(
        body,
        grid=(N // B,),
        in_specs=[
            pl.BlockSpec((2, B, H), lambda i: (0, i, 0)),
            pl.BlockSpec((B, F), lambda i: (i, 0)),
            pl.BlockSpec((B, 1), lambda i: (i, 0)),
            pl.BlockSpec((B, 1), lambda i: (i, 0)),
            pl.BlockSpec((F, 512), lambda i: (0, 0)),
            pl.BlockSpec((1, 512), lambda i: (0, 0)),
            pl.BlockSpec((512, F), lambda i: (0, 0)),
        ],
        out_specs=[
            pl.BlockSpec((B, F), lambda i: (i, 0)),
            pl.BlockSpec((2, B, H), lambda i: (0, i, 0)),
        ],
        out_shape=[
            jax.ShapeDtypeStruct((N, F), jnp.float32),
            jax.ShapeDtypeStruct((2, N, H), jnp.float32),
        ],
    )(raw1, x, dis, inv, W1, b1, W2)


def _tc_decode(raw2, z, dis, inv, b2, Wd, bd):
    def body(raw_ref, z_ref, dis_ref, inv_ref, b2_ref, Wd_ref, bd_ref,
             lp_ref, emb_ref):
        out2 = (jnp.concatenate([raw_ref[0], raw_ref[1]], axis=1)
                * dis_ref[...] + z_ref[...] * inv_ref[...] + b2_ref[...])
        nrm = jnp.sqrt(jnp.sum(out2 * out2, axis=1, keepdims=True))
        emb = out2 / (nrm + 1e-12)
        emb_ref[...] = emb
        logits = (jnp.dot(emb, Wd_ref[...], preferred_element_type=jnp.float32)
                  + bd_ref[...])
        m = jnp.max(logits, axis=1, keepdims=True)
        lse = m + jnp.log(jnp.sum(jnp.exp(logits - m), axis=1, keepdims=True))
        lp_ref[...] = logits - lse

    return pl.pallas_call(
        body,
        grid=(N // B,),
        in_specs=[
            pl.BlockSpec((2, B, H), lambda i: (0, i, 0)),
            pl.BlockSpec((B, F), lambda i: (i, 0)),
            pl.BlockSpec((B, 1), lambda i: (i, 0)),
            pl.BlockSpec((B, 1), lambda i: (i, 0)),
            pl.BlockSpec((1, F), lambda i: (0, 0)),
            pl.BlockSpec((F, 128), lambda i: (0, 0)),
            pl.BlockSpec((1, 128), lambda i: (0, 0)),
        ],
        out_specs=[
            pl.BlockSpec((B, 128), lambda i: (i, 0)),
            pl.BlockSpec((B, F), lambda i: (i, 0)),
        ],
        out_shape=[
            jax.ShapeDtypeStruct((N, 128), jnp.float32),
            jax.ShapeDtypeStruct((N, F), jnp.float32),
        ],
    )(raw2, z, dis, inv, b2, Wd, bd)


def kernel(x, edge_index, W1, b1, W2, b2, Wd, bd):
    npad = CPSX * C - EPS
    srcr = edge_index[0].reshape(NS, EPS)
    dstr = edge_index[1].reshape(NS, EPS)
    gpad = jnp.broadcast_to((jnp.arange(npad, dtype=jnp.int32) * 131) % N,
                            (NS, npad))
    dpad = jnp.broadcast_to(N + (jnp.arange(npad, dtype=jnp.int32) % NTR),
                            (NS, npad))
    g0 = jnp.concatenate([srcr, gpad], axis=1)
    g1 = jnp.concatenate([srcr + N, gpad], axis=1)
    dd = jnp.concatenate([dstr, dpad], axis=1)
    eidx = jnp.stack([g0, g1, dd]).reshape(3, NS, CPSX, 1, C)
    raw1, _, dinfo = _sc_layer1(x, eidx)
    dis = dinfo[0].reshape(NPAD, 1)[:N]
    inv = dinfo[1].reshape(NPAD, 1)[:N]
    z, zs_cat = _tc_mlp(raw1, x, dis, inv, W1, b1.reshape(1, -1), W2)
    raw2 = _sc_aggregate(zs_cat.reshape(2 * N, H), eidx)
    lp, emb = _tc_decode(raw2, z, dis, inv, b2.reshape(1, -1), Wd,
                         bd.reshape(1, -1))
    return (lp, emb)


# speedup vs baseline: 1.0134x; 1.0003x over previous
import functools

import jax
import jax.numpy as jnp
from jax import lax
from jax.experimental import pallas as pl
from jax.experimental.pallas import tpu as pltpu
from jax.experimental.pallas import tpu_sc as plsc

N = 10000
E = 160000
F = 256
H = 128
NC = 2
NS = 16
NW = NC * NS
C = 128
NCHUNK = E // C
NPAD = 10240
RPS = N // NS
B = 1000

EPS = E // NS
CPSP = 80
CPSX = CPSP + 1
PADE = CPSP * C - EPS
NTR = 8


def _mesh():
    return plsc.VectorSubcoreMesh(
        core_axis_name="c", subcore_axis_name="s",
        num_cores=NC, num_subcores=NS)


SEG = NPAD // NS


def _sc_layer1(x, eidx):
    @functools.partial(
        pl.kernel,
        out_type=[
            jax.ShapeDtypeStruct((NC, N, H), jnp.float32),
            jax.ShapeDtypeStruct((NC * N, H), jnp.float32),
            jax.ShapeDtypeStruct((2, NS, 1, SEG), jnp.float32),
        ],
        mesh=_mesh(),
        scratch_types=[
            pltpu.VMEM_SHARED((N + NTR, H), jnp.float32),
            pltpu.VMEM_SHARED((NPAD,), jnp.float32),
            pltpu.VMEM((CPSX, 1, C), jnp.int32),
            [pltpu.VMEM((C,), jnp.int32) for _ in range(2)],
            [pltpu.VMEM((C, H), jnp.float32) for _ in range(2)],
            pltpu.VMEM((C,), jnp.float32),
            pltpu.VMEM((SEG,), jnp.float32),
            pltpu.VMEM((SEG,), jnp.float32),
            pltpu.VMEM((SEG,), jnp.float32),
            [pltpu.SemaphoreType.DMA for _ in range(2)],
            [pltpu.SemaphoreType.DMA for _ in range(2)],
            [pltpu.SemaphoreType.DMA for _ in range(2)],
            pltpu.SemaphoreType.DMA,
        ],
    )
    def l1(x_hbm, eidx_hbm, raw_hbm, xs_hbm, dinfo_hbm, accum, counts,
           didx, gidx, rows, ones_v, cnt_v, dis_v, inv_v,
           sem_g, sem_s, sem_i, sem_h):
        c = lax.axis_index("c")
        s = lax.axis_index("s")
        zero16 = jnp.zeros((16,), jnp.float32)
        one16 = jnp.ones((16,), jnp.float32)

        pltpu.sync_copy(eidx_hbm.at[2, s], didx)

        def ob(i, _):
            ones_v[pl.ds(i * 16, 16)] = one16
            return 0
        lax.fori_loop(0, C // 16, ob, 0)

        def zcb(i, _):
            cnt_v[pl.ds(i * 16, 16)] = zero16
            return 0
        lax.fori_loop(0, SEG // 16, zcb, 0)
        pltpu.sync_copy(cnt_v, counts.at[pl.ds(s * SEG, SEG)])
        plsc.subcore_barrier()

        def hb(t, _):
            pltpu.async_copy(ones_v, counts.at[didx.at[t, 0]], sem_h,
                             add=True)
            return 0
        lax.fori_loop(0, CPSP, hb, 0)

        def zb(t, _):
            rows[0][t // (H // 16), pl.ds((t % (H // 16)) * 16, 16)] = zero16
            return 0
        lax.fori_loop(0, C * (H // 16), zb, 0)
        r0 = s * RPS
        for q in range(RPS // C):
            pltpu.sync_copy(rows[0], accum.at[pl.ds(r0 + q * C, C)])
        pltpu.sync_copy(rows[0].at[pl.ds(0, RPS % C)],
                        accum.at[pl.ds(r0 + (RPS // C) * C, RPS % C)])

        @pl.when(s == 0)
        def _ztrash():
            pltpu.sync_copy(rows[0].at[pl.ds(0, NTR)],
                            accum.at[pl.ds(N, NTR)])

        def hd(t, _):
            pltpu.make_async_copy(ones_v, counts.at[pl.ds(0, C)],
                                  sem_h).wait()
            return 0
        lax.fori_loop(0, CPSP, hd, 0)
        plsc.subcore_barrier()

        pltpu.sync_copy(counts.at[pl.ds(s * SEG, SEG)], cnt_v)

        def db(i, _):
            deg = cnt_v[pl.ds(i * 16, 16)] + 1.0
            inv = 1.0 / deg
            inv_v[pl.ds(i * 16, 16)] = inv
            sq = 0.5 * (deg + 1.0)
            for _it in range(15):
                sq = 0.5 * (sq + deg / sq)
            dis_v[pl.ds(i * 16, 16)] = inv * sq
            return 0
        lax.fori_loop(0, SEG // 16, db, 0)
        pltpu.sync_copy(dis_v, dinfo_hbm.at[0, s, 0])
        pltpu.sync_copy(inv_v, dinfo_hbm.at[1, s, 0])

        def scale_block(nb, cnt):
            g0 = s * SEG + nb
            pltpu.sync_copy(x_hbm.at[pl.ds(g0, cnt), pl.ds(c * H, H)],
                            rows[0].at[pl.ds(0, cnt)])

            dnums = lax.GatherDimensionNumbers(
                offset_dims=(), collapsed_slice_dims=(0,),
                start_index_map=(0,))

            def grp_body(gi, _):
                dis16 = dis_v[pl.ds(nb + gi * 16, 16)]
                for j in range(16):
                    dsp = lax.gather(
                        dis16, jnp.full((16, 1), j, jnp.int32), dnums, (1,),
                        mode=lax.GatherScatterMode.PROMISE_IN_BOUNDS)
                    i = gi * 16 + j
                    for f in range(H // 16):
                        rows[0][i, pl.ds(f * 16, 16)] = (
                            rows[0][i, pl.ds(f * 16, 16)] * dsp)
                return 0
            lax.fori_loop(0, cnt // 16, grp_body, 0)
            pltpu.sync_copy(rows[0].at[pl.ds(0, cnt)],
                            xs_hbm.at[pl.ds(c * N + g0, cnt)])

        @pl.when(s < NS - 1)
        def _scale_full():
            for q in range(SEG // C):
                scale_block(q * C, C)

        @pl.when(s == NS - 1)
        def _scale_tail():
            nlast = N - SEG * (NS - 1)
            for q in range(nlast // C):
                scale_block(q * C, C)
            scale_block((nlast // C) * C, nlast % C)

        plsc.subcore_barrier()

        def fire_i(k, t):
            pltpu.async_copy(eidx_hbm.at[c, s, t, 0], gidx[k], sem_i[k])

        def wait_i(k):
            pltpu.make_async_copy(eidx_hbm.at[c, s, 0, 0], gidx[k],
                                  sem_i[k]).wait()

        def fire_g(k):
            pltpu.async_copy(xs_hbm.at[gidx[k]], rows[k], sem_g[k])

        def wait_g(k):
            pltpu.make_async_copy(xs_hbm.at[pl.ds(0, C)], rows[k],
                                  sem_g[k]).wait()

        def fire_s(k, t):
            pltpu.async_copy(rows[k], accum.at[didx.at[t, 0]], sem_s[k],
                             add=True)

        def wait_s(k):
            pltpu.make_async_copy(rows[k], accum.at[pl.ds(0, C)],
                                  sem_s[k]).wait()

        fire_i(0, 0)
        fire_i(1, 1)
        wait_i(0)
        fire_g(0)
        wait_i(1)
        fire_g(1)
        wait_g(0)
        fire_s(0, 0)
        fire_i(0, 2)

        def step(t, k):
            wait_i(k)
            wait_s(k)
            fire_g(k)
            wait_g(1 - k)
            fire_s(1 - k, t - 1)
            fire_i(1 - k, t + 1)

        def body(i, _):
            t = 2 * i + 2
            step(t, 0)
            step(t + 1, 1)
            return 0
        lax.fori_loop(0, (CPSP - 2) // 2, body, 0)

        wait_g(1)
        fire_s(1, CPSP - 1)
        wait_s(0)
        wait_s(1)
        wait_i(0)

        plsc.subcore_barrier()
        o0 = s * SEG

        @pl.when(s < NS - 1)
        def _copy_main():
            pltpu.sync_copy(accum.at[pl.ds(o0, SEG)],
                            raw_hbm.at[c, pl.ds(o0, SEG)])

        @pl.when(s == NS - 1)
        def _copy_tail():
            pltpu.sync_copy(accum.at[pl.ds(o0, N - SEG * (NS - 1))],
                            raw_hbm.at[c, pl.ds(o0, N - SEG * (NS - 1))])

    return l1(x, eidx)


def _sc_aggregate(table2n, eidx):
    @functools.partial(
        pl.kernel,
        out_type=jax.ShapeDtypeStruct((NC, N, H), jnp.float32),
        mesh=_mesh(),
        scratch_types=[
            pltpu.VMEM_SHARED((N + NTR, H), jnp.float32),
            pltpu.VMEM((CPSX, 1, C), jnp.int32),
            [pltpu.VMEM((C,), jnp.int32) for _ in range(2)],
            [pltpu.VMEM((C, H), jnp.float32) for _ in range(2)],
            [pltpu.SemaphoreType.DMA for _ in range(2)],
            [pltpu.SemaphoreType.DMA for _ in range(2)],
            [pltpu.SemaphoreType.DMA for _ in range(2)],
        ],
    )
    def agg(tab_hbm, eidx_hbm, out_hbm, accum, didx, gidx, rows,
            sem_g, sem_s, sem_i):
        c = lax.axis_index("c")
        s = lax.axis_index("s")
        zero16 = jnp.zeros((16,), jnp.float32)

        pltpu.sync_copy(eidx_hbm.at[2, s], didx)

        def zb(t, _):
            rows[0][t // (H // 16), pl.ds((t % (H // 16)) * 16, 16)] = zero16
            return 0
        lax.fori_loop(0, C * (H // 16), zb, 0)
        r0 = s * RPS
        for q in range(RPS // C):
            pltpu.sync_copy(rows[0], accum.at[pl.ds(r0 + q * C, C)])
        pltpu.sync_copy(rows[0].at[pl.ds(0, RPS % C)],
                        accum.at[pl.ds(r0 + (RPS // C) * C, RPS % C)])
        @pl.when(s == 0)
        def _ztrash():
            pltpu.sync_copy(rows[0].at[pl.ds(0, NTR)],
                            accum.at[pl.ds(N, NTR)])

        def fire_i(k, t):
            pltpu.async_copy(eidx_hbm.at[c, s, t, 0], gidx[k], sem_i[k])

        def wait_i(k):
            pltpu.make_async_copy(eidx_hbm.at[c, s, 0, 0], gidx[k],
                                  sem_i[k]).wait()

        def fire_g(k):
            pltpu.async_copy(tab_hbm.at[gidx[k]], rows[k], sem_g[k])

        def wait_g(k):
            pltpu.make_async_copy(tab_hbm.at[pl.ds(0, C)], rows[k],
                                  sem_g[k]).wait()

        def fire_s(k, t):
            pltpu.async_copy(rows[k], accum.at[didx.at[t, 0]], sem_s[k],
                             add=True)

        def wait_s(k):
            pltpu.make_async_copy(rows[k], accum.at[pl.ds(0, C)],
                                  sem_s[k]).wait()

        fire_i(0, 0)
        fire_i(1, 1)
        plsc.subcore_barrier()
        wait_i(0)
        fire_g(0)
        wait_i(1)
        fire_g(1)
        wait_g(0)
        fire_s(0, 0)
        fire_i(0, 2)

        def step(t, k):
            wait_i(k)
            wait_s(k)
            fire_g(k)
            wait_g(1 - k)
            fire_s(1 - k, t - 1)
            fire_i(1 - k, t + 1)

        def body(i, _):
            t = 2 * i + 2
            step(t, 0)
            step(t + 1, 1)
            return 0
        lax.fori_loop(0, (CPSP - 2) // 2, body, 0)

        wait_g(1)
        fire_s(1, CPSP - 1)
        wait_s(0)
        wait_s(1)
        wait_i(0)

        plsc.subcore_barrier()
        o0 = s * 640

        @pl.when(s < NS - 1)
        def _copy_main():
            pltpu.sync_copy(accum.at[pl.ds(o0, 640)],
                            out_hbm.at[c, pl.ds(o0, 640)])

        @pl.when(s == NS - 1)
        def _copy_tail():
            pltpu.sync_copy(accum.at[pl.ds(o0, N - 640 * (NS - 1))],
                            out_hbm.at[c, pl.ds(o0, N - 640 * (NS - 1))])

    return agg(table2n, eidx)


def _tc_mlp(raw1, x, dis, inv, W1, b1, W2):
    def body(raw_ref, x_ref, dis_ref, inv_ref, W1_ref, b1_ref, W2_ref,
             z_ref, zs_ref):
        dis = dis_ref[...]
        agg = (jnp.concatenate([raw_ref[0], raw_ref[1]], axis=1) * dis
               + x_ref[...] * inv_ref[...])
        h = jnp.maximum(
            jnp.dot(agg, W1_ref[...], preferred_element_type=jnp.float32)
            + b1_ref[...], 0.0)
        z = jnp.dot(h, W2_ref[...], preferred_element_type=jnp.float32)
        z_ref[...] = z
        zs = z * dis
        zs_ref[0] = zs[:, :H]
        zs_ref[1] = zs[:, H:]

    return pl.pallas_call(
        body,
        grid=(N // B,),
        in_specs=[
            pl.BlockSpec((2, B, H), lambda i: (0, i, 0)),
            pl.BlockSpec((B, F), lambda i: (i, 0)),
            pl.BlockSpec((B, 1), lambda i: (i, 0)),
            pl.BlockSpec((B, 1), lambda i: (i, 0)),
            pl.BlockSpec((F, 512), lambda i: (0, 0)),
            pl.BlockSpec((1, 512), lambda i: (0, 0)),
            pl.BlockSpec((512, F), lambda i: (0, 0)),
        ],
        out_specs=[
            pl.BlockSpec((B, F), lambda i: (i, 0)),
            pl.BlockSpec((2, B, H), lambda i: (0, i, 0)),
        ],
        out_shape=[
            jax.ShapeDtypeStruct((N, F), jnp.float32),
            jax.ShapeDtypeStruct((2, N, H), jnp.float32),
        ],
    )(raw1, x, dis, inv, W1, b1, W2)


def _tc_decode(raw2, z, dis, inv, b2, Wd, bd):
    def body(raw_ref, z_ref, dis_ref, inv_ref, b2_ref, Wd_ref, bd_ref,
             lp_ref, emb_ref):
        out2 = (jnp.concatenate([raw_ref[0], raw_ref[1]], axis=1)
                * dis_ref[...] + z_ref[...] * inv_ref[...] + b2_ref[...])
        nrm = jnp.sqrt(jnp.sum(out2 * out2, axis=1, keepdims=True))
        emb = out2 / (nrm + 1e-12)
        emb_ref[...] = emb
        logits = (jnp.dot(emb, Wd_ref[...], preferred_element_type=jnp.float32)
                  + bd_ref[...])
        m = jnp.max(logits, axis=1, keepdims=True)
        lse = m + jnp.log(jnp.sum(jnp.exp(logits - m), axis=1, keepdims=True))
        lp_ref[...] = logits - lse

    return pl.pallas_call(
        body,
        grid=(N // B,),
        in_specs=[
            pl.BlockSpec((2, B, H), lambda i: (0, i, 0)),
            pl.BlockSpec((B, F), lambda i: (i, 0)),
            pl.BlockSpec((B, 1), lambda i: (i, 0)),
            pl.BlockSpec((B, 1), lambda i: (i, 0)),
            pl.BlockSpec((1, F), lambda i: (0, 0)),
            pl.BlockSpec((F, 128), lambda i: (0, 0)),
            pl.BlockSpec((1, 128), lambda i: (0, 0)),
        ],
        out_specs=[
            pl.BlockSpec((B, 128), lambda i: (i, 0)),
            pl.BlockSpec((B, F), lambda i: (i, 0)),
        ],
        out_shape=[
            jax.ShapeDtypeStruct((N, 128), jnp.float32),
            jax.ShapeDtypeStruct((N, F), jnp.float32),
        ],
    )(raw2, z, dis, inv, b2, Wd, bd)


def kernel(x, edge_index, W1, b1, W2, b2, Wd, bd):
    npad = CPSX * C - EPS
    srcr = edge_index[0].reshape(NS, EPS)
    dstr = edge_index[1].reshape(NS, EPS)
    gpad = jnp.broadcast_to((jnp.arange(npad, dtype=jnp.int32) * 131) % N,
                            (NS, npad))
    dpad = jnp.broadcast_to(N + (jnp.arange(npad, dtype=jnp.int32) % NTR),
                            (NS, npad))
    g0 = jnp.concatenate([srcr, gpad], axis=1)
    g1 = jnp.concatenate([srcr + N, gpad], axis=1)
    dd = jnp.concatenate([dstr, dpad], axis=1)
    eidx = jnp.stack([g0, g1, dd]).reshape(3, NS, CPSX, 1, C)
    raw1, _, dinfo = _sc_layer1(x, eidx)
    dis = dinfo[0].reshape(NPAD, 1)[:N]
    inv = dinfo[1].reshape(NPAD, 1)[:N]
    z, zs_cat = _tc_mlp(raw1, x, dis, inv, W1, b1.reshape(1, -1), W2)
    raw2 = _sc_aggregate(zs_cat.reshape(2 * N, H), eidx)
    lp, emb = _tc_decode(raw2, z, dis, inv, b2.reshape(1, -1), Wd,
                         bd.reshape(1, -1))
    return (lp, emb)


# speedup vs baseline: 1.0204x; 1.0069x over previous
import functools

import jax
import jax.numpy as jnp
from jax import lax
from jax.experimental import pallas as pl
from jax.experimental.pallas import tpu as pltpu
from jax.experimental.pallas import tpu_sc as plsc

N = 10000
E = 160000
F = 256
H = 128
NC = 2
NS = 16
NW = NC * NS
C = 128
NCHUNK = E // C
NPAD = 10240
RPS = N // NS
B = 1000

EPS = E // NS
CPSP = 80
CPSX = CPSP + 1
PADE = CPSP * C - EPS
NTR = 8


def _mesh():
    return plsc.VectorSubcoreMesh(
        core_axis_name="c", subcore_axis_name="s",
        num_cores=NC, num_subcores=NS)


def _sc_hist(eidx):
    HCH = CPSP // NC

    @functools.partial(
        pl.kernel,
        out_type=jax.ShapeDtypeStruct((NC, NPAD), jnp.float32),
        mesh=_mesh(),
        scratch_types=[
            pltpu.VMEM_SHARED((NPAD,), jnp.float32),
            pltpu.VMEM((HCH, 1, C), jnp.int32),
            pltpu.VMEM((C,), jnp.float32),
            pltpu.VMEM((NPAD // NS,), jnp.float32),
            pltpu.SemaphoreType.DMA,
        ],
    )
    def hist(eidx_hbm, out_hbm, counts, didx, ones_v, zeros_v, sem):
        c = lax.axis_index("c")
        s = lax.axis_index("s")
        zero16 = jnp.zeros((16,), jnp.float32)
        one16 = jnp.ones((16,), jnp.float32)

        pltpu.sync_copy(eidx_hbm.at[2, s, pl.ds(c * HCH, HCH)], didx)

        def ob(i, _):
            ones_v[pl.ds(i * 16, 16)] = one16
            return 0
        lax.fori_loop(0, C // 16, ob, 0)

        def zb(i, _):
            zeros_v[pl.ds(i * 16, 16)] = zero16
            return 0
        lax.fori_loop(0, (NPAD // NS) // 16, zb, 0)

        seg = NPAD // NS
        pltpu.sync_copy(zeros_v, counts.at[pl.ds(s * seg, seg)])
        plsc.subcore_barrier()

        def body(t, _):
            pltpu.async_copy(ones_v, counts.at[didx.at[t, 0]], sem, add=True)
            return 0
        lax.fori_loop(0, HCH, body, 0)

        def drain(t, _):
            pltpu.make_async_copy(ones_v, counts.at[pl.ds(0, C)], sem).wait()
            return 0
        lax.fori_loop(0, HCH, drain, 0)

        plsc.subcore_barrier()
        pltpu.sync_copy(counts.at[pl.ds(s * seg, seg)],
                        out_hbm.at[c, pl.ds(s * seg, seg)])

    return hist(eidx)


def _sc_aggregate(table2n, eidx):
    @functools.partial(
        pl.kernel,
        out_type=jax.ShapeDtypeStruct((NC, N, H), jnp.float32),
        mesh=_mesh(),
        scratch_types=[
            pltpu.VMEM_SHARED((N + NTR, H), jnp.float32),
            pltpu.VMEM((CPSX, 1, C), jnp.int32),
            [pltpu.VMEM((C,), jnp.int32) for _ in range(2)],
            [pltpu.VMEM((C, H), jnp.float32) for _ in range(2)],
            [pltpu.SemaphoreType.DMA for _ in range(2)],
            [pltpu.SemaphoreType.DMA for _ in range(2)],
            [pltpu.SemaphoreType.DMA for _ in range(2)],
        ],
    )
    def agg(tab_hbm, eidx_hbm, out_hbm, accum, didx, gidx, rows,
            sem_g, sem_s, sem_i):
        c = lax.axis_index("c")
        s = lax.axis_index("s")
        zero16 = jnp.zeros((16,), jnp.float32)

        pltpu.sync_copy(eidx_hbm.at[2, s], didx)

        def zb(t, _):
            rows[0][t // (H // 16), pl.ds((t % (H // 16)) * 16, 16)] = zero16
            return 0
        lax.fori_loop(0, C * (H // 16), zb, 0)
        r0 = s * RPS
        for q in range(RPS // C):
            pltpu.sync_copy(rows[0], accum.at[pl.ds(r0 + q * C, C)])
        pltpu.sync_copy(rows[0].at[pl.ds(0, RPS % C)],
                        accum.at[pl.ds(r0 + (RPS // C) * C, RPS % C)])
        @pl.when(s == 0)
        def _ztrash():
            pltpu.sync_copy(rows[0].at[pl.ds(0, NTR)],
                            accum.at[pl.ds(N, NTR)])

        def fire_i(k, t):
            pltpu.async_copy(eidx_hbm.at[c, s, t, 0], gidx[k], sem_i[k])

        def wait_i(k):
            pltpu.make_async_copy(eidx_hbm.at[c, s, 0, 0], gidx[k],
                                  sem_i[k]).wait()

        def fire_g(k):
            pltpu.async_copy(tab_hbm.at[gidx[k]], rows[k], sem_g[k])

        def wait_g(k):
            pltpu.make_async_copy(tab_hbm.at[pl.ds(0, C)], rows[k],
                                  sem_g[k]).wait()

        def fire_s(k, t):
            pltpu.async_copy(rows[k], accum.at[didx.at[t, 0]], sem_s[k],
                             add=True)

        def wait_s(k):
            pltpu.make_async_copy(rows[k], accum.at[pl.ds(0, C)],
                                  sem_s[k]).wait()

        fire_i(0, 0)
        fire_i(1, 1)
        plsc.subcore_barrier()
        wait_i(0)
        fire_g(0)
        wait_i(1)
        fire_g(1)
        wait_g(0)
        fire_s(0, 0)
        fire_i(0, 2)

        def step(t, k):
            wait_i(k)
            wait_s(k)
            fire_g(k)
            wait_g(1 - k)
            fire_s(1 - k, t - 1)
            fire_i(1 - k, t + 1)

        def body(i, _):
            t = 2 * i + 2
            step(t, 0)
            step(t + 1, 1)
            return 0
        lax.fori_loop(0, (CPSP - 2) // 2, body, 0)

        wait_g(1)
        fire_s(1, CPSP - 1)
        wait_s(0)
        wait_s(1)
        wait_i(0)

        plsc.subcore_barrier()
        o0 = s * 640

        @pl.when(s < NS - 1)
        def _copy_main():
            pltpu.sync_copy(accum.at[pl.ds(o0, 640)],
                            out_hbm.at[c, pl.ds(o0, 640)])

        @pl.when(s == NS - 1)
        def _copy_tail():
            pltpu.sync_copy(accum.at[pl.ds(o0, N - 640 * (NS - 1))],
                            out_hbm.at[c, pl.ds(o0, N - 640 * (NS - 1))])

    return agg(table2n, eidx)


def _tc_scale(countT, x):
    def body(cnt_ref, x_ref, xs_ref, dis_ref, inv_ref):
        deg = cnt_ref[:, 0:1] + cnt_ref[:, 1:2] + 1.0
        dis = lax.rsqrt(deg)
        dis_ref[...] = dis
        inv_ref[...] = 1.0 / deg
        xs = x_ref[...] * dis
        xs_ref[0] = xs[:, :H]
        xs_ref[1] = xs[:, H:]

    return pl.pallas_call(
        body,
        grid=(N // B,),
        in_specs=[
            pl.BlockSpec((B, 2), lambda i: (i, 0)),
            pl.BlockSpec((B, F), lambda i: (i, 0)),
        ],
        out_specs=[
            pl.BlockSpec((2, B, H), lambda i: (0, i, 0)),
            pl.BlockSpec((B, 1), lambda i: (i, 0)),
            pl.BlockSpec((B, 1), lambda i: (i, 0)),
        ],
        out_shape=[
            jax.ShapeDtypeStruct((2, N, H), jnp.float32),
            jax.ShapeDtypeStruct((N, 1), jnp.float32),
            jax.ShapeDtypeStruct((N, 1), jnp.float32),
        ],
    )(countT, x)


def _tc_mlp(raw1, x, dis, inv, W1, b1, W2):
    def body(raw_ref, x_ref, dis_ref, inv_ref, W1_ref, b1_ref, W2_ref,
             z_ref, zs_ref):
        dis = dis_ref[...]
        agg = (jnp.concatenate([raw_ref[0], raw_ref[1]], axis=1) * dis
               + x_ref[...] * inv_ref[...])
        h = jnp.maximum(
            jnp.dot(agg, W1_ref[...], preferred_element_type=jnp.float32)
            + b1_ref[...], 0.0)
        z = jnp.dot(h, W2_ref[...], preferred_element_type=jnp.float32)
        z_ref[...] = z
        zs = z * dis
        zs_ref[0] = zs[:, :H]
        zs_ref[1] = zs[:, H:]

    return pl.pallas_call(
        body,
        grid=(N // B,),
        in_specs=[
            pl.BlockSpec((2, B, H), lambda i: (0, i, 0)),
            pl.BlockSpec((B, F), lambda i: (i, 0)),
            pl.BlockSpec((B, 1), lambda i: (i, 0)),
            pl.BlockSpec((B, 1), lambda i: (i, 0)),
            pl.BlockSpec((F, 512), lambda i: (0, 0)),
            pl.BlockSpec((1, 512), lambda i: (0, 0)),
            pl.BlockSpec((512, F), lambda i: (0, 0)),
        ],
        out_specs=[
            pl.BlockSpec((B, F), lambda i: (i, 0)),
            pl.BlockSpec((2, B, H), lambda i: (0, i, 0)),
        ],
        out_shape=[
            jax.ShapeDtypeStruct((N, F), jnp.float32),
            jax.ShapeDtypeStruct((2, N, H), jnp.float32),
        ],
    )(raw1, x, dis, inv, W1, b1, W2)


def _tc_decode(raw2, z, dis, inv, b2, Wd, bd):
    def body(raw_ref, z_ref, dis_ref, inv_ref, b2_ref, Wd_ref, bd_ref,
             lp_ref, emb_ref):
        out2 = (jnp.concatenate([raw_ref[0], raw_ref[1]], axis=1)
                * dis_ref[...] + z_ref[...] * inv_ref[...] + b2_ref[...])
        nrm = jnp.sqrt(jnp.sum(out2 * out2, axis=1, keepdims=True))
        emb = out2 / (nrm + 1e-12)
        emb_ref[...] = emb
        logits = (jnp.dot(emb, Wd_ref[...], preferred_element_type=jnp.float32)
                  + bd_ref[...])
        m = jnp.max(logits, axis=1, keepdims=True)
        lse = m + jnp.log(jnp.sum(jnp.exp(logits - m), axis=1, keepdims=True))
        lp_ref[...] = logits - lse

    return pl.pallas_call(
        body,
        grid=(N // B,),
        in_specs=[
            pl.BlockSpec((2, B, H), lambda i: (0, i, 0)),
            pl.BlockSpec((B, F), lambda i: (i, 0)),
            pl.BlockSpec((B, 1), lambda i: (i, 0)),
            pl.BlockSpec((B, 1), lambda i: (i, 0)),
            pl.BlockSpec((1, F), lambda i: (0, 0)),
            pl.BlockSpec((F, 128), lambda i: (0, 0)),
            pl.BlockSpec((1, 128), lambda i: (0, 0)),
        ],
        out_specs=[
            pl.BlockSpec((B, 128), lambda i: (i, 0)),
            pl.BlockSpec((B, F), lambda i: (i, 0)),
        ],
        out_shape=[
            jax.ShapeDtypeStruct((N, 128), jnp.float32),
            jax.ShapeDtypeStruct((N, F), jnp.float32),
        ],
    )(raw2, z, dis, inv, b2, Wd, bd)


def kernel(x, edge_index, W1, b1, W2, b2, Wd, bd):
    npad = CPSX * C - EPS
    srcr = edge_index[0].reshape(NS, EPS)
    dstr = edge_index[1].reshape(NS, EPS)
    gpad = jnp.broadcast_to((jnp.arange(npad, dtype=jnp.int32) * 131) % N,
                            (NS, npad))
    dpad = jnp.broadcast_to(N + (jnp.arange(npad, dtype=jnp.int32) % NTR),
                            (NS, npad))
    g0 = jnp.concatenate([srcr, gpad], axis=1)
    g1 = jnp.concatenate([srcr + N, gpad], axis=1)
    dd = jnp.concatenate([dstr, dpad], axis=1)
    eidx = jnp.stack([g0, g1, dd]).reshape(3, NS, CPSX, 1, C)
    counts = _sc_hist(eidx)
    countT = jnp.transpose(counts[:, :N])
    xs_cat, dis, inv = _tc_scale(countT, x)
    raw1 = _sc_aggregate(xs_cat.reshape(2 * N, H), eidx)
    z, zs_cat = _tc_mlp(raw1, x, dis, inv, W1, b1.reshape(1, -1), W2)
    raw2 = _sc_aggregate(zs_cat.reshape(2 * N, H), eidx)
    lp, emb = _tc_decode(raw2, z, dis, inv, b2.reshape(1, -1), Wd,
                         bd.reshape(1, -1))
    return (lp, emb)


# speedup vs baseline: 1.0303x; 1.0097x over previous
import functools

import jax
import jax.numpy as jnp
from jax import lax
from jax.experimental import pallas as pl
from jax.experimental.pallas import tpu as pltpu
from jax.experimental.pallas import tpu_sc as plsc

N = 10000
E = 160000
F = 256
H = 128
NC = 2
NS = 16
NW = NC * NS
C = 128
NCHUNK = E // C
NPAD = 10240
RPS = N // NS
B = 1000

EPS = E // NS
CPSP = 80
CPSX = CPSP + 1
PADE = CPSP * C - EPS
NTR = 8


def _mesh():
    return plsc.VectorSubcoreMesh(
        core_axis_name="c", subcore_axis_name="s",
        num_cores=NC, num_subcores=NS)


def _sc_hist(eidx):
    HCH = CPSP // NC

    @functools.partial(
        pl.kernel,
        out_type=jax.ShapeDtypeStruct((NC, NPAD), jnp.float32),
        mesh=_mesh(),
        scratch_types=[
            pltpu.VMEM_SHARED((NPAD,), jnp.float32),
            pltpu.VMEM((HCH, 1, C), jnp.int32),
            pltpu.VMEM((C,), jnp.float32),
            pltpu.VMEM((NPAD // NS,), jnp.float32),
            pltpu.SemaphoreType.DMA,
        ],
    )
    def hist(eidx_hbm, out_hbm, counts, didx, ones_v, zeros_v, sem):
        c = lax.axis_index("c")
        s = lax.axis_index("s")
        zero16 = jnp.zeros((16,), jnp.float32)
        one16 = jnp.ones((16,), jnp.float32)

        pltpu.sync_copy(eidx_hbm.at[2, s, pl.ds(c * HCH, HCH)], didx)

        def ob(i, _):
            ones_v[pl.ds(i * 16, 16)] = one16
            return 0
        lax.fori_loop(0, C // 16, ob, 0)

        def zb(i, _):
            zeros_v[pl.ds(i * 16, 16)] = zero16
            return 0
        lax.fori_loop(0, (NPAD // NS) // 16, zb, 0)

        seg = NPAD // NS
        pltpu.sync_copy(zeros_v, counts.at[pl.ds(s * seg, seg)])
        plsc.subcore_barrier()

        def body(t, _):
            pltpu.async_copy(ones_v, counts.at[didx.at[t, 0]], sem, add=True)
            return 0
        lax.fori_loop(0, HCH, body, 0)

        def drain(t, _):
            pltpu.make_async_copy(ones_v, counts.at[pl.ds(0, C)], sem).wait()
            return 0
        lax.fori_loop(0, HCH, drain, 0)

        plsc.subcore_barrier()
        pltpu.sync_copy(counts.at[pl.ds(s * seg, seg)],
                        out_hbm.at[c, pl.ds(s * seg, seg)])

    return hist(eidx)


def _sc_aggregate(table2n, eidx):
    @functools.partial(
        pl.kernel,
        out_type=jax.ShapeDtypeStruct((NC, N, H), jnp.float32),
        mesh=_mesh(),
        scratch_types=[
            pltpu.VMEM_SHARED((N + NTR, H), jnp.float32),
            pltpu.VMEM((CPSX, 1, C), jnp.int32),
            [pltpu.VMEM((C,), jnp.int32) for _ in range(2)],
            [pltpu.VMEM((C, H), jnp.float32) for _ in range(2)],
            [pltpu.SemaphoreType.DMA for _ in range(2)],
            [pltpu.SemaphoreType.DMA for _ in range(2)],
            [pltpu.SemaphoreType.DMA for _ in range(2)],
            pltpu.SemaphoreType.DMA,
        ],
    )
    def agg(tab_hbm, eidx_hbm, out_hbm, accum, didx, gidx, rows,
            sem_g, sem_s, sem_i, sem_d):
        c = lax.axis_index("c")
        s = lax.axis_index("s")
        zero16 = jnp.zeros((16,), jnp.float32)

        pltpu.async_copy(eidx_hbm.at[2, s], didx, sem_d)

        def fire_i(k, t):
            pltpu.async_copy(eidx_hbm.at[c, s, t, 0], gidx[k], sem_i[k])

        def wait_i(k):
            pltpu.make_async_copy(eidx_hbm.at[c, s, 0, 0], gidx[k],
                                  sem_i[k]).wait()

        def fire_g(k):
            pltpu.async_copy(tab_hbm.at[gidx[k]], rows[k], sem_g[k])

        def wait_g(k):
            pltpu.make_async_copy(tab_hbm.at[pl.ds(0, C)], rows[k],
                                  sem_g[k]).wait()

        def fire_s(k, t):
            pltpu.async_copy(rows[k], accum.at[didx.at[t, 0]], sem_s[k],
                             add=True)

        def wait_s(k):
            pltpu.make_async_copy(rows[k], accum.at[pl.ds(0, C)],
                                  sem_s[k]).wait()

        fire_i(0, 0)
        fire_i(1, 1)
        wait_i(1)
        fire_g(1)

        def zb(t, _):
            rows[0][t // (H // 16), pl.ds((t % (H // 16)) * 16, 16)] = zero16
            return 0
        lax.fori_loop(0, C * (H // 16), zb, 0)
        r0 = s * RPS
        for q in range(RPS // C):
            pltpu.sync_copy(rows[0], accum.at[pl.ds(r0 + q * C, C)])
        pltpu.sync_copy(rows[0].at[pl.ds(0, RPS % C)],
                        accum.at[pl.ds(r0 + (RPS // C) * C, RPS % C)])
        @pl.when(s == 0)
        def _ztrash():
            pltpu.sync_copy(rows[0].at[pl.ds(0, NTR)],
                            accum.at[pl.ds(N, NTR)])

        wait_i(0)
        fire_g(0)
        pltpu.make_async_copy(eidx_hbm.at[2, s], didx, sem_d).wait()
        plsc.subcore_barrier()
        wait_g(0)
        fire_s(0, 0)
        fire_i(0, 2)

        def step(t, k):
            wait_i(k)
            wait_s(k)
            fire_g(k)
            wait_g(1 - k)
            fire_s(1 - k, t - 1)
            fire_i(1 - k, t + 1)

        def body(i, _):
            t = 2 * i + 2
            step(t, 0)
            step(t + 1, 1)
            return 0
        lax.fori_loop(0, (CPSP - 2) // 2, body, 0)

        wait_g(1)
        fire_s(1, CPSP - 1)
        wait_s(0)
        wait_s(1)
        wait_i(0)

        plsc.subcore_barrier()
        o0 = s * 640

        @pl.when(s < NS - 1)
        def _copy_main():
            pltpu.sync_copy(accum.at[pl.ds(o0, 640)],
                            out_hbm.at[c, pl.ds(o0, 640)])

        @pl.when(s == NS - 1)
        def _copy_tail():
            pltpu.sync_copy(accum.at[pl.ds(o0, N - 640 * (NS - 1))],
                            out_hbm.at[c, pl.ds(o0, N - 640 * (NS - 1))])

    return agg(table2n, eidx)


def _tc_scale(countT, x):
    def body(cnt_ref, x_ref, xs_ref, dis_ref, inv_ref):
        deg = cnt_ref[:, 0:1] + cnt_ref[:, 1:2] + 1.0
        dis = lax.rsqrt(deg)
        dis_ref[...] = dis
        inv_ref[...] = 1.0 / deg
        xs = x_ref[...] * dis
        xs_ref[0] = xs[:, :H]
        xs_ref[1] = xs[:, H:]

    return pl.pallas_call(
        body,
        grid=(N // B,),
        in_specs=[
            pl.BlockSpec((B, 2), lambda i: (i, 0)),
            pl.BlockSpec((B, F), lambda i: (i, 0)),
        ],
        out_specs=[
            pl.BlockSpec((2, B, H), lambda i: (0, i, 0)),
            pl.BlockSpec((B, 1), lambda i: (i, 0)),
            pl.BlockSpec((B, 1), lambda i: (i, 0)),
        ],
        out_shape=[
            jax.ShapeDtypeStruct((2, N, H), jnp.float32),
            jax.ShapeDtypeStruct((N, 1), jnp.float32),
            jax.ShapeDtypeStruct((N, 1), jnp.float32),
        ],
    )(countT, x)


def _tc_mlp(raw1, x, dis, inv, W1, b1, W2):
    def body(raw_ref, x_ref, dis_ref, inv_ref, W1_ref, b1_ref, W2_ref,
             z_ref, zs_ref):
        dis = dis_ref[...]
        agg = (jnp.concatenate([raw_ref[0], raw_ref[1]], axis=1) * dis
               + x_ref[...] * inv_ref[...])
        h = jnp.maximum(
            jnp.dot(agg, W1_ref[...], preferred_element_type=jnp.float32)
            + b1_ref[...], 0.0)
        z = jnp.dot(h, W2_ref[...], preferred_element_type=jnp.float32)
        z_ref[...] = z
        zs = z * dis
        zs_ref[0] = zs[:, :H]
        zs_ref[1] = zs[:, H:]

    return pl.pallas_call(
        body,
        grid=(N // B,),
        in_specs=[
            pl.BlockSpec((2, B, H), lambda i: (0, i, 0)),
            pl.BlockSpec((B, F), lambda i: (i, 0)),
            pl.BlockSpec((B, 1), lambda i: (i, 0)),
            pl.BlockSpec((B, 1), lambda i: (i, 0)),
            pl.BlockSpec((F, 512), lambda i: (0, 0)),
            pl.BlockSpec((1, 512), lambda i: (0, 0)),
            pl.BlockSpec((512, F), lambda i: (0, 0)),
        ],
        out_specs=[
            pl.BlockSpec((B, F), lambda i: (i, 0)),
            pl.BlockSpec((2, B, H), lambda i: (0, i, 0)),
        ],
        out_shape=[
            jax.ShapeDtypeStruct((N, F), jnp.float32),
            jax.ShapeDtypeStruct((2, N, H), jnp.float32),
        ],
    )(raw1, x, dis, inv, W1, b1, W2)


def _tc_decode(raw2, z, dis, inv, b2, Wd, bd):
    def body(raw_ref, z_ref, dis_ref, inv_ref, b2_ref, Wd_ref, bd_ref,
             lp_ref, emb_ref):
        out2 = (jnp.concatenate([raw_ref[0], raw_ref[1]], axis=1)
                * dis_ref[...] + z_ref[...] * inv_ref[...] + b2_ref[...])
        nrm = jnp.sqrt(jnp.sum(out2 * out2, axis=1, keepdims=True))
        emb = out2 / (nrm + 1e-12)
        emb_ref[...] = emb
        logits = (jnp.dot(emb, Wd_ref[...], preferred_element_type=jnp.float32)
                  + bd_ref[...])
        m = jnp.max(logits, axis=1, keepdims=True)
        lse = m + jnp.log(jnp.sum(jnp.exp(logits - m), axis=1, keepdims=True))
        lp_ref[...] = logits - lse

    return pl.pallas_call(
        body,
        grid=(N // B,),
        in_specs=[
            pl.BlockSpec((2, B, H), lambda i: (0, i, 0)),
            pl.BlockSpec((B, F), lambda i: (i, 0)),
            pl.BlockSpec((B, 1), lambda i: (i, 0)),
            pl.BlockSpec((B, 1), lambda i: (i, 0)),
            pl.BlockSpec((1, F), lambda i: (0, 0)),
            pl.BlockSpec((F, 128), lambda i: (0, 0)),
            pl.BlockSpec((1, 128), lambda i: (0, 0)),
        ],
        out_specs=[
            pl.BlockSpec((B, 128), lambda i: (i, 0)),
            pl.BlockSpec((B, F), lambda i: (i, 0)),
        ],
        out_shape=[
            jax.ShapeDtypeStruct((N, 128), jnp.float32),
            jax.ShapeDtypeStruct((N, F), jnp.float32),
        ],
    )(raw2, z, dis, inv, b2, Wd, bd)


def kernel(x, edge_index, W1, b1, W2, b2, Wd, bd):
    npad = CPSX * C - EPS
    srcr = edge_index[0].reshape(NS, EPS)
    dstr = edge_index[1].reshape(NS, EPS)
    gpad = jnp.broadcast_to((jnp.arange(npad, dtype=jnp.int32) * 131) % N,
                            (NS, npad))
    dpad = jnp.broadcast_to(N + (jnp.arange(npad, dtype=jnp.int32) % NTR),
                            (NS, npad))
    g0 = jnp.concatenate([srcr, gpad], axis=1)
    g1 = jnp.concatenate([srcr + N, gpad], axis=1)
    dd = jnp.concatenate([dstr, dpad], axis=1)
    eidx = jnp.stack([g0, g1, dd]).reshape(3, NS, CPSX, 1, C)
    counts = _sc_hist(eidx)
    countT = jnp.transpose(counts[:, :N])
    xs_cat, dis, inv = _tc_scale(countT, x)
    raw1 = _sc_aggregate(xs_cat.reshape(2 * N, H), eidx)
    z, zs_cat = _tc_mlp(raw1, x, dis, inv, W1, b1.reshape(1, -1), W2)
    raw2 = _sc_aggregate(zs_cat.reshape(2 * N, H), eidx)
    lp, emb = _tc_decode(raw2, z, dis, inv, b2.reshape(1, -1), Wd,
                         bd.reshape(1, -1))
    return (lp, emb)
